# Initial kernel scaffold; baseline (speedup 1.0000x reference)
import functools

import jax
import jax.numpy as jnp
from jax import lax
from jax.experimental import pallas as pl
from jax.experimental.pallas import tpu as pltpu
from jax.experimental.pallas import tpu_sc as plsc

_N = 10000
_H = 5000
_D = 128

_NC = 2
_NS = 16
_NW = _NC * _NS
_K = 128

_NPAD = 10240
_HPAD = 5120

_BR = 1024


def _prep_edges(senders, receivers, conv, epad):
    e = senders.shape[0]
    nb = epad // (_NW * _K)
    pad = epad - e
    s = jnp.pad(senders, (0, pad)).reshape(_NW, nb, _K)
    r = jnp.pad(receivers, (0, pad)).reshape(_NW, nb, _K)
    c = jnp.pad(conv.reshape(-1), (0, pad)).reshape(_NW, nb * _K)
    return s, r, c


def _init_zero_bufs(zrow, zc, cbuf):
    z = jnp.zeros((16,), jnp.float32)

    def row(k, carry):
        for j in range(8):
            zrow[k, pl.ds(j * 16, 16)] = z
        zc[k, pl.ds(0, 16)] = z
        cbuf[k, pl.ds(0, 16)] = z
        return carry

    lax.fori_loop(0, _K, row, 0)


def _zero_slice(zrow, zc, acc_a, acc_c, r0, rows):
    done = 0
    while done < rows:
        cnt = min(_K, rows - done)
        pltpu.sync_copy(zrow.at[pl.ds(0, cnt)], acc_a.at[pl.ds(r0 + done, cnt)])
        pltpu.sync_copy(zc.at[pl.ds(0, cnt)], acc_c.at[pl.ds(r0 + done, cnt)])
        done += cnt


def _run_stage(wid, cid, sid, nb, s_in, r_in, c_in, table, acc_a, acc_c,
               out_a, out_c, vpad, sidx, ridx, cval, rows, cbuf, zrow, zc, sem):
    rpt = vpad // _NS
    r0 = sid * rpt

    pltpu.sync_copy(s_in.at[wid], sidx.at[pl.ds(0, nb)])
    pltpu.sync_copy(r_in.at[wid], ridx.at[pl.ds(0, nb)])
    pltpu.sync_copy(c_in.at[wid], cval.at[pl.ds(0, nb * _K)])

    _zero_slice(zrow, zc, acc_a, acc_c, r0, rpt)
    plsc.subcore_barrier()

    lane = lax.iota(jnp.int32, 16)
    col0 = jnp.zeros((16,), jnp.int32)

    def batch(b, carry):
        pltpu.async_copy(table.at[sidx.at[b]], rows, sem).wait()

        def scale(k, carry2):
            ck = cval[b * _K + k]
            for j in range(8):
                sl = pl.ds(j * 16, 16)
                rows[k, sl] = rows[k, sl] * ck
            return carry2

        lax.fori_loop(0, _K, scale, 0)

        for g in range(8):
            vals = cval[pl.ds(b * _K + g * 16, 16)]
            plsc.store_scatter(cbuf, [lane + g * 16, col0], vals)

        pltpu.sync_copy(rows, acc_a.at[ridx.at[b]], add=True)
        pltpu.sync_copy(cbuf, acc_c.at[ridx.at[b]], add=True)
        return carry

    lax.fori_loop(0, nb, batch, 0)
    plsc.subcore_barrier()

    pltpu.sync_copy(acc_a.at[pl.ds(r0, rpt)], out_a.at[cid, pl.ds(r0, rpt)])
    pltpu.sync_copy(acc_c.at[pl.ds(r0, rpt)], out_c.at[cid, pl.ds(r0, rpt)])
    plsc.subcore_barrier()


def _sc_kernel_nodes(nodes, hedges, s_nn, r_nn, c_nn, s_h2, r_h2, c_h2,
                     s_hh, r_hh, c_hh,
                     a_nn, c1_nn, a_h2, c1_h2, a_hh, c1_hh,
                     sidx, ridx, cval, rows, cbuf, zrow, zc, acc_a, acc_c, sem):
    cid = lax.axis_index("c")
    sid = lax.axis_index("s")
    wid = cid * _NS + sid
    _init_zero_bufs(zrow, zc, cbuf)
    nb_en = s_nn.shape[1]
    nb_eh = s_hh.shape[1]
    _run_stage(wid, cid, sid, nb_en, s_nn, r_nn, c_nn, nodes, acc_a, acc_c,
               a_nn, c1_nn, _NPAD, sidx, ridx, cval, rows, cbuf, zrow, zc, sem)
    _run_stage(wid, cid, sid, nb_en, s_h2, r_h2, c_h2, hedges, acc_a, acc_c,
               a_h2, c1_h2, _NPAD, sidx, ridx, cval, rows, cbuf, zrow, zc, sem)
    _run_stage(wid, cid, sid, nb_eh, s_hh, r_hh, c_hh, hedges, acc_a, acc_c,
               a_hh, c1_hh, _HPAD, sidx, ridx, cval, rows, cbuf, zrow, zc, sem)


def _sc_kernel_n2h(node_table, s_nh, r_nh, c_nh, a_nh, c1_nh,
                   sidx, ridx, cval, rows, cbuf, zrow, zc, acc_a, acc_c, sem):
    cid = lax.axis_index("c")
    sid = lax.axis_index("s")
    wid = cid * _NS + sid
    _init_zero_bufs(zrow, zc, cbuf)
    nb = s_nh.shape[1]
    _run_stage(wid, cid, sid, nb, s_nh, r_nh, c_nh, node_table, acc_a, acc_c,
               a_nh, c1_nh, _HPAD, sidx, ridx, cval, rows, cbuf, zrow, zc, sem)


def _make_sc_nodes(nb_en, nb_eh):
    mesh = plsc.VectorSubcoreMesh(core_axis_name="c", subcore_axis_name="s",
                                  num_cores=_NC, num_subcores=_NS)
    f32 = jnp.float32
    out_type = (
        jax.ShapeDtypeStruct((_NC, _NPAD, _D), f32),
        jax.ShapeDtypeStruct((_NC, _NPAD, 16), f32),
        jax.ShapeDtypeStruct((_NC, _NPAD, _D), f32),
        jax.ShapeDtypeStruct((_NC, _NPAD, 16), f32),
        jax.ShapeDtypeStruct((_NC, _HPAD, _D), f32),
        jax.ShapeDtypeStruct((_NC, _HPAD, 16), f32),
    )
    scratch = [
        pltpu.VMEM((nb_en, _K), jnp.int32),
        pltpu.VMEM((nb_en, _K), jnp.int32),
        pltpu.VMEM((nb_en * _K,), f32),
        pltpu.VMEM((_K, _D), f32),
        pltpu.VMEM((_K, 16), f32),
        pltpu.VMEM((_K, _D), f32),
        pltpu.VMEM((_K, 16), f32),
        pltpu.VMEM_SHARED((_NPAD, _D), f32),
        pltpu.VMEM_SHARED((_NPAD, 16), f32),
        pltpu.SemaphoreType.DMA,
    ]
    return pl.kernel(_sc_kernel_nodes, out_type=out_type, mesh=mesh,
                     scratch_types=scratch)


def _make_sc_n2h(nb):
    mesh = plsc.VectorSubcoreMesh(core_axis_name="c", subcore_axis_name="s",
                                  num_cores=_NC, num_subcores=_NS)
    f32 = jnp.float32
    out_type = (
        jax.ShapeDtypeStruct((_NC, _HPAD, _D), f32),
        jax.ShapeDtypeStruct((_NC, _HPAD, 16), f32),
    )
    scratch = [
        pltpu.VMEM((nb, _K), jnp.int32),
        pltpu.VMEM((nb, _K), jnp.int32),
        pltpu.VMEM((nb * _K,), f32),
        pltpu.VMEM((_K, _D), f32),
        pltpu.VMEM((_K, 16), f32),
        pltpu.VMEM((_K, _D), f32),
        pltpu.VMEM((_K, 16), f32),
        pltpu.VMEM_SHARED((_HPAD, _D), f32),
        pltpu.VMEM_SHARED((_HPAD, 16), f32),
        pltpu.SemaphoreType.DMA,
    ]
    return pl.kernel(_sc_kernel_n2h, out_type=out_type, mesh=mesh,
                     scratch_types=scratch)


def _tc_combine_body(a_m, c_m, a_s, c_s, wt_m, b_m, wt_s, b_s, out):
    am = a_m[0] + a_m[1]
    cm = (c_m[0] + c_m[1])[:, 0:1]
    gm = jnp.dot(am, wt_m[...], preferred_element_type=jnp.float32)
    gm = gm + cm * b_m[...]
    as_ = a_s[0] + a_s[1]
    cs = (c_s[0] + c_s[1])[:, 0:1]
    gs = jnp.dot(as_, wt_s[...], preferred_element_type=jnp.float32)
    gs = gs + cs * b_s[...]
    out[...] = jnp.tanh(gs * gm)


def _tc_combine(a_m, c_m, a_s, c_s, wt_m, b_m, wt_s, b_s, vpad):
    grid = vpad // _BR
    f32 = jnp.float32
    blk_a = pl.BlockSpec((_NC, _BR, _D), lambda i: (0, i, 0))
    blk_c = pl.BlockSpec((_NC, _BR, 16), lambda i: (0, i, 0))
    blk_w = pl.BlockSpec((_D, _D), lambda i: (0, 0))
    blk_b = pl.BlockSpec((1, _D), lambda i: (0, 0))
    return pl.pallas_call(
        _tc_combine_body,
        grid=(grid,),
        in_specs=[blk_a, blk_c, blk_a, blk_c, blk_w, blk_b, blk_w, blk_b],
        out_specs=pl.BlockSpec((_BR, _D), lambda i: (i, 0)),
        out_shape=jax.ShapeDtypeStruct((vpad, _D), f32),
    )(a_m, c_m, a_s, c_s, wt_m, b_m, wt_s, b_s)


def kernel(node_features, hedge_features, node_senders, node_receivers,
           node_convolution, hedge2node_senders, hedge2node_receivers,
           hedge2node_convolution, hedge_senders, hedge_receivers,
           hedge_convolution, node2hedge_senders, node2hedge_receivers,
           node2hedge_convolution, W_nm, b_nm, W_hs, b_hs, W_hm, b_hm,
           W_ns, b_ns):
    en = node_senders.shape[0]
    eh = hedge_senders.shape[0]
    chunk = _NW * _K
    en_pad = -(-en // chunk) * chunk
    eh_pad = -(-eh // chunk) * chunk

    s_nn, r_nn, c_nn = _prep_edges(node_senders, node_receivers,
                                   node_convolution, en_pad)
    s_h2, r_h2, c_h2 = _prep_edges(hedge2node_senders, hedge2node_receivers,
                                   hedge2node_convolution, en_pad)
    s_hh, r_hh, c_hh = _prep_edges(hedge_senders, hedge_receivers,
                                   hedge_convolution, eh_pad)
    s_nh, r_nh, c_nh = _prep_edges(node2hedge_senders, node2hedge_receivers,
                                   node2hedge_convolution, en_pad)

    sc_nodes = _make_sc_nodes(en_pad // chunk, eh_pad // chunk)
    a_nn, c1_nn, a_h2, c1_h2, a_hh, c1_hh = sc_nodes(
        node_features, hedge_features, s_nn, r_nn, c_nn,
        s_h2, r_h2, c_h2, s_hh, r_hh, c_hh)

    new_node_full = _tc_combine(a_h2, c1_h2, a_nn, c1_nn,
                                W_hs.T, b_hs.reshape(1, _D),
                                W_nm.T, b_nm.reshape(1, _D), _NPAD)

    sc_n2h = _make_sc_n2h(en_pad // chunk)
    a_nh, c1_nh = sc_n2h(new_node_full, s_nh, r_nh, c_nh)

    new_hedge_full = _tc_combine(a_nh, c1_nh, a_hh, c1_hh,
                                 W_ns.T, b_ns.reshape(1, _D),
                                 W_hm.T, b_hm.reshape(1, _D), _HPAD)

    return new_node_full[:_N], new_hedge_full[:_H]


# speedup vs baseline: 2.0152x; 2.0152x over previous
import jax
import jax.numpy as jnp
from jax import lax
from jax.experimental import pallas as pl
from jax.experimental.pallas import tpu as pltpu
from jax.experimental.pallas import tpu_sc as plsc

_N = 10000
_H = 5000
_D = 128

_NC = 2
_NS = 16
_NW = _NC * _NS
_K = 128
_CH = 8

_NPAD = 10240
_HPAD = 5120

_BR = 1024


def _prep_edges(senders, receivers, conv):
    e = senders.shape[0]
    chunk = _NW * _K * _CH
    epad = -(-e // chunk) * chunk
    nb = epad // (_NW * _K)
    pad = epad - e
    s = jnp.pad(senders, (0, pad)).reshape(_NW, nb, _K)
    r = jnp.pad(receivers, (0, pad)).reshape(_NW, nb, _K)
    c = jnp.pad(conv.reshape(-1), (0, pad)).reshape(_NW, nb, _K)
    return s, r, c


def _run_stage(wid, cid, sid, nchunks, s_in, r_in, c_in, table, acc_a, acc_c,
               out_a, out_c, vpad, sidx, ridx, cvalb, rows, cbuf, sem):
    rpt = vpad // _NS
    r0 = sid * rpt
    z = jnp.zeros((16,), jnp.float32)
    lane = lax.iota(jnp.int32, 16)

    def zero_row(k, carry):
        for j in range(8):
            rows[k, pl.ds(j * 16, 16)] = z
        cbuf[k, pl.ds(0, 16)] = z
        return carry

    lax.fori_loop(0, _K, zero_row, 0)

    done = 0
    while done < rpt:
        cnt = min(_K, rpt - done)
        pltpu.sync_copy(rows.at[pl.ds(0, cnt)], acc_a.at[pl.ds(r0 + done, cnt)])
        pltpu.sync_copy(cbuf.at[pl.ds(0, cnt)], acc_c.at[pl.ds(r0 + done, cnt)])
        done += cnt
    plsc.subcore_barrier()

    def chunk_body(ch, carry):
        c0 = ch * _CH
        pltpu.sync_copy(s_in.at[wid, pl.ds(c0, _CH)], sidx)
        pltpu.sync_copy(r_in.at[wid, pl.ds(c0, _CH)], ridx)
        pltpu.sync_copy(c_in.at[wid, pl.ds(c0, _CH)], cvalb)

        def batch(j, carry2):
            pltpu.async_copy(table.at[sidx.at[j]], rows, sem).wait()

            def scale(g, carry3):
                cw = cvalb[j, pl.ds(g * 16, 16)]
                for l in range(16):
                    ck = cw[l]
                    k = g * 16 + l
                    for f in range(8):
                        sl = pl.ds(f * 16, 16)
                        rows[k, sl] = rows[k, sl] * ck
                    cbuf[k, pl.ds(0, 16)] = jnp.where(lane == 0, ck, 0.0)
                return carry3

            lax.fori_loop(0, 8, scale, 0)

            pltpu.sync_copy(rows, acc_a.at[ridx.at[j]], add=True)
            pltpu.sync_copy(cbuf, acc_c.at[ridx.at[j]], add=True)
            return carry2

        lax.fori_loop(0, _CH, batch, 0)
        return carry

    lax.fori_loop(0, nchunks, chunk_body, 0)
    plsc.subcore_barrier()

    pltpu.sync_copy(acc_a.at[pl.ds(r0, rpt)], out_a.at[cid, pl.ds(r0, rpt)])
    pltpu.sync_copy(acc_c.at[pl.ds(r0, rpt)], out_c.at[cid, pl.ds(r0, rpt)])
    plsc.subcore_barrier()


def _sc_kernel_nodes(nodes, hedges, s_nn, r_nn, c_nn, s_h2, r_h2, c_h2,
                     s_hh, r_hh, c_hh,
                     a_nn, c1_nn, a_h2, c1_h2, a_hh, c1_hh,
                     sidx, ridx, cvalb, rows, cbuf, acc_a, acc_c, sem):
    cid = lax.axis_index("c")
    sid = lax.axis_index("s")
    wid = cid * _NS + sid
    _run_stage(wid, cid, sid, s_nn.shape[1] // _CH, s_nn, r_nn, c_nn, nodes,
               acc_a, acc_c, a_nn, c1_nn, _NPAD, sidx, ridx, cvalb, rows,
               cbuf, sem)
    _run_stage(wid, cid, sid, s_h2.shape[1] // _CH, s_h2, r_h2, c_h2, hedges,
               acc_a, acc_c, a_h2, c1_h2, _NPAD, sidx, ridx, cvalb, rows,
               cbuf, sem)
    _run_stage(wid, cid, sid, s_hh.shape[1] // _CH, s_hh, r_hh, c_hh, hedges,
               acc_a, acc_c, a_hh, c1_hh, _HPAD, sidx, ridx, cvalb, rows,
               cbuf, sem)


def _sc_kernel_n2h(node_table, s_nh, r_nh, c_nh, a_nh, c1_nh,
                   sidx, ridx, cvalb, rows, cbuf, acc_a, acc_c, sem):
    cid = lax.axis_index("c")
    sid = lax.axis_index("s")
    wid = cid * _NS + sid
    _run_stage(wid, cid, sid, s_nh.shape[1] // _CH, s_nh, r_nh, c_nh,
               node_table, acc_a, acc_c, a_nh, c1_nh, _HPAD, sidx, ridx,
               cvalb, rows, cbuf, sem)


def _sc_scratch(vpad):
    f32 = jnp.float32
    return [
        pltpu.VMEM((_CH, _K), jnp.int32),
        pltpu.VMEM((_CH, _K), jnp.int32),
        pltpu.VMEM((_CH, _K), f32),
        pltpu.VMEM((_K, _D), f32),
        pltpu.VMEM((_K, 16), f32),
        pltpu.VMEM_SHARED((vpad, _D), f32),
        pltpu.VMEM_SHARED((vpad, 16), f32),
        pltpu.SemaphoreType.DMA,
    ]


def _make_sc_nodes():
    mesh = plsc.VectorSubcoreMesh(core_axis_name="c", subcore_axis_name="s",
                                  num_cores=_NC, num_subcores=_NS)
    f32 = jnp.float32
    out_type = (
        jax.ShapeDtypeStruct((_NC, _NPAD, _D), f32),
        jax.ShapeDtypeStruct((_NC, _NPAD, 16), f32),
        jax.ShapeDtypeStruct((_NC, _NPAD, _D), f32),
        jax.ShapeDtypeStruct((_NC, _NPAD, 16), f32),
        jax.ShapeDtypeStruct((_NC, _HPAD, _D), f32),
        jax.ShapeDtypeStruct((_NC, _HPAD, 16), f32),
    )
    return pl.kernel(_sc_kernel_nodes, out_type=out_type, mesh=mesh,
                     scratch_types=_sc_scratch(_NPAD),
                     compiler_params=pltpu.CompilerParams(
                         use_tc_tiling_on_sc=False))


def _make_sc_n2h():
    mesh = plsc.VectorSubcoreMesh(core_axis_name="c", subcore_axis_name="s",
                                  num_cores=_NC, num_subcores=_NS)
    f32 = jnp.float32
    out_type = (
        jax.ShapeDtypeStruct((_NC, _HPAD, _D), f32),
        jax.ShapeDtypeStruct((_NC, _HPAD, 16), f32),
    )
    return pl.kernel(_sc_kernel_n2h, out_type=out_type, mesh=mesh,
                     scratch_types=_sc_scratch(_HPAD),
                     compiler_params=pltpu.CompilerParams(
                         use_tc_tiling_on_sc=False))


def _tc_combine_body(a_m, c_m, a_s, c_s, wt_m, b_m, wt_s, b_s, out):
    am = a_m[0] + a_m[1]
    cm = (c_m[0] + c_m[1])[:, 0:1]
    gm = jnp.dot(am, wt_m[...], preferred_element_type=jnp.float32,
                 precision=lax.Precision.HIGHEST)
    gm = gm + cm * b_m[...]
    as_ = a_s[0] + a_s[1]
    cs = (c_s[0] + c_s[1])[:, 0:1]
    gs = jnp.dot(as_, wt_s[...], preferred_element_type=jnp.float32,
                 precision=lax.Precision.HIGHEST)
    gs = gs + cs * b_s[...]
    out[...] = jnp.tanh(gs * gm)


def _tc_combine(a_m, c_m, a_s, c_s, wt_m, b_m, wt_s, b_s, vpad):
    grid = vpad // _BR
    blk_a = pl.BlockSpec((_NC, _BR, _D), lambda i: (0, i, 0))
    blk_c = pl.BlockSpec((_NC, _BR, 16), lambda i: (0, i, 0))
    blk_w = pl.BlockSpec((_D, _D), lambda i: (0, 0))
    blk_b = pl.BlockSpec((1, _D), lambda i: (0, 0))
    return pl.pallas_call(
        _tc_combine_body,
        grid=(grid,),
        in_specs=[blk_a, blk_c, blk_a, blk_c, blk_w, blk_b, blk_w, blk_b],
        out_specs=pl.BlockSpec((_BR, _D), lambda i: (i, 0)),
        out_shape=jax.ShapeDtypeStruct((vpad, _D), jnp.float32),
    )(a_m, c_m, a_s, c_s, wt_m, b_m, wt_s, b_s)


def kernel(node_features, hedge_features, node_senders, node_receivers,
           node_convolution, hedge2node_senders, hedge2node_receivers,
           hedge2node_convolution, hedge_senders, hedge_receivers,
           hedge_convolution, node2hedge_senders, node2hedge_receivers,
           node2hedge_convolution, W_nm, b_nm, W_hs, b_hs, W_hm, b_hm,
           W_ns, b_ns):
    s_nn, r_nn, c_nn = _prep_edges(node_senders, node_receivers,
                                   node_convolution)
    s_h2, r_h2, c_h2 = _prep_edges(hedge2node_senders, hedge2node_receivers,
                                   hedge2node_convolution)
    s_hh, r_hh, c_hh = _prep_edges(hedge_senders, hedge_receivers,
                                   hedge_convolution)
    s_nh, r_nh, c_nh = _prep_edges(node2hedge_senders, node2hedge_receivers,
                                   node2hedge_convolution)

    a_nn, c1_nn, a_h2, c1_h2, a_hh, c1_hh = _make_sc_nodes()(
        node_features, hedge_features, s_nn, r_nn, c_nn,
        s_h2, r_h2, c_h2, s_hh, r_hh, c_hh)

    new_node_full = _tc_combine(a_h2, c1_h2, a_nn, c1_nn,
                                W_hs.T, b_hs.reshape(1, _D),
                                W_nm.T, b_nm.reshape(1, _D), _NPAD)

    a_nh, c1_nh = _make_sc_n2h()(new_node_full, s_nh, r_nh, c_nh)

    new_hedge_full = _tc_combine(a_nh, c1_nh, a_hh, c1_hh,
                                 W_ns.T, b_ns.reshape(1, _D),
                                 W_hm.T, b_hm.reshape(1, _D), _HPAD)

    return new_node_full[:_N], new_hedge_full[:_H]


# speedup vs baseline: 2.5664x; 1.2735x over previous
import jax
import jax.numpy as jnp
from jax import lax
from jax.experimental import pallas as pl
from jax.experimental.pallas import tpu as pltpu
from jax.experimental.pallas import tpu_sc as plsc

_N = 10000
_H = 5000
_D = 128

_NC = 2
_NS = 16
_NW = _NC * _NS
_K = 128

_NPAD = 10240
_HPAD = 5120

_BR = 1024


def _prep_edges(senders, receivers, conv):
    e = senders.shape[0]
    chunk = _NW * _K * 2
    epad = -(-e // chunk) * chunk
    nb = epad // (_NW * _K)
    pad = epad - e
    s = jnp.pad(senders, (0, pad)).reshape(_NW, nb, _K)
    r = jnp.pad(receivers, (0, pad)).reshape(_NW, nb, _K)
    c = jnp.pad(conv.reshape(-1), (0, pad)).reshape(_NW, nb, _K)
    return jnp.stack([s, r], axis=2), c


def _scale_rows(cv, rows, cbuf, lane):
    def body(g, carry):
        cw = cv[pl.ds(g * 16, 16)]
        for l in range(16):
            ck = cw[l]
            k = g * 16 + l
            for f in range(8):
                sl = pl.ds(f * 16, 16)
                rows[k, sl] = rows[k, sl] * ck
            cbuf[k, pl.ds(0, 16)] = jnp.where(lane == 0, ck, 0.0)
        return carry

    lax.fori_loop(0, 8, body, 0)


def _copy_ridx(ebuf, ridx):
    for g in range(8):
        sl = pl.ds(g * 16, 16)
        ridx[sl] = ebuf[1, sl]


def _run_stage(wid, cid, sid, nb, e_in, c_in, table, acc_a, acc_c, out_a,
               out_c, vpad, ebA, ebB, riA, riB, cvA, cvB, rwA, rwB, cbuf,
               gsA, gsB, ssA, ssB, esA, esB, csA, csB):
    rpt = vpad // _NS
    r0 = sid * rpt
    z = jnp.zeros((16,), jnp.float32)
    lane = lax.iota(jnp.int32, 16)
    npair = nb // 2

    def zero_row(k, carry):
        for j in range(8):
            rwA[k, pl.ds(j * 16, 16)] = z
        cbuf[k, pl.ds(0, 16)] = z
        return carry

    lax.fori_loop(0, _K, zero_row, 0)

    done = 0
    while done < rpt:
        cnt = min(_K, rpt - done)
        pltpu.sync_copy(rwA.at[pl.ds(0, cnt)], acc_a.at[pl.ds(r0 + done, cnt)])
        pltpu.sync_copy(cbuf.at[pl.ds(0, cnt)], acc_c.at[pl.ds(r0 + done, cnt)])
        done += cnt
    plsc.subcore_barrier()

    pltpu.sync_copy(e_in.at[wid, 0], ebA)
    pltpu.sync_copy(c_in.at[wid, 0], cvA)
    pltpu.async_copy(table.at[ebA.at[0]], rwA, gsA)
    pltpu.async_copy(e_in.at[wid, 1], ebB, esB)
    pltpu.async_copy(c_in.at[wid, 1], cvB, csB)

    def pair(i, carry):
        pltpu.make_async_copy(table.at[ebA.at[0]], rwA, gsA).wait()

        @pl.when(i > 0)
        def _():
            pltpu.make_async_copy(rwB, acc_a.at[riB], ssB).wait()

        pltpu.make_async_copy(e_in.at[wid, 2 * i + 1], ebB, esB).wait()
        pltpu.make_async_copy(c_in.at[wid, 2 * i + 1], cvB, csB).wait()
        pltpu.async_copy(table.at[ebB.at[0]], rwB, gsB)
        _copy_ridx(ebA, riA)
        _scale_rows(cvA, rwA, cbuf, lane)
        pltpu.async_copy(rwA, acc_a.at[riA], ssA, add=True)
        pltpu.sync_copy(cbuf, acc_c.at[riA], add=True)

        @pl.when(i < npair - 1)
        def _():
            pltpu.async_copy(e_in.at[wid, 2 * i + 2], ebA, esA)
            pltpu.async_copy(c_in.at[wid, 2 * i + 2], cvA, csA)

        pltpu.make_async_copy(table.at[ebB.at[0]], rwB, gsB).wait()
        pltpu.make_async_copy(rwA, acc_a.at[riA], ssA).wait()

        @pl.when(i < npair - 1)
        def _():
            pltpu.make_async_copy(e_in.at[wid, 2 * i + 2], ebA, esA).wait()
            pltpu.make_async_copy(c_in.at[wid, 2 * i + 2], cvA, csA).wait()
            pltpu.async_copy(table.at[ebA.at[0]], rwA, gsA)

        _copy_ridx(ebB, riB)
        _scale_rows(cvB, rwB, cbuf, lane)
        pltpu.async_copy(rwB, acc_a.at[riB], ssB, add=True)
        pltpu.sync_copy(cbuf, acc_c.at[riB], add=True)

        @pl.when(i < npair - 1)
        def _():
            pltpu.async_copy(e_in.at[wid, 2 * i + 3], ebB, esB)
            pltpu.async_copy(c_in.at[wid, 2 * i + 3], cvB, csB)

        return carry

    lax.fori_loop(0, npair, pair, 0)
    pltpu.make_async_copy(rwB, acc_a.at[riB], ssB).wait()
    plsc.subcore_barrier()

    pltpu.sync_copy(acc_a.at[pl.ds(r0, rpt)], out_a.at[cid, pl.ds(r0, rpt)])
    pltpu.sync_copy(acc_c.at[pl.ds(r0, rpt)], out_c.at[cid, pl.ds(r0, rpt)])
    plsc.subcore_barrier()


def _sc_kernel_nodes(nodes, hedges, e_nn, c_nn, e_h2, c_h2, e_hh, c_hh,
                     a_nn, c1_nn, a_h2, c1_h2, a_hh, c1_hh,
                     ebA, ebB, riA, riB, cvA, cvB, rwA, rwB, cbuf,
                     acc_a, acc_c,
                     gsA, gsB, ssA, ssB, esA, esB, csA, csB):
    cid = lax.axis_index("c")
    sid = lax.axis_index("s")
    wid = cid * _NS + sid
    args = (ebA, ebB, riA, riB, cvA, cvB, rwA, rwB, cbuf,
            gsA, gsB, ssA, ssB, esA, esB, csA, csB)
    _run_stage(wid, cid, sid, e_nn.shape[1], e_nn, c_nn, nodes, acc_a, acc_c,
               a_nn, c1_nn, _NPAD, *args)
    _run_stage(wid, cid, sid, e_h2.shape[1], e_h2, c_h2, hedges, acc_a, acc_c,
               a_h2, c1_h2, _NPAD, *args)
    _run_stage(wid, cid, sid, e_hh.shape[1], e_hh, c_hh, hedges, acc_a, acc_c,
               a_hh, c1_hh, _HPAD, *args)


def _sc_kernel_n2h(node_table, e_nh, c_nh, a_nh, c1_nh,
                   ebA, ebB, riA, riB, cvA, cvB, rwA, rwB, cbuf,
                   acc_a, acc_c,
                   gsA, gsB, ssA, ssB, esA, esB, csA, csB):
    cid = lax.axis_index("c")
    sid = lax.axis_index("s")
    wid = cid * _NS + sid
    args = (ebA, ebB, riA, riB, cvA, cvB, rwA, rwB, cbuf,
            gsA, gsB, ssA, ssB, esA, esB, csA, csB)
    _run_stage(wid, cid, sid, e_nh.shape[1], e_nh, c_nh, node_table,
               acc_a, acc_c, a_nh, c1_nh, _HPAD, *args)


def _sc_scratch(vpad):
    f32 = jnp.float32
    return [
        pltpu.VMEM((2, _K), jnp.int32),
        pltpu.VMEM((2, _K), jnp.int32),
        pltpu.VMEM((_K,), jnp.int32),
        pltpu.VMEM((_K,), jnp.int32),
        pltpu.VMEM((_K,), f32),
        pltpu.VMEM((_K,), f32),
        pltpu.VMEM((_K, _D), f32),
        pltpu.VMEM((_K, _D), f32),
        pltpu.VMEM((_K, 16), f32),
        pltpu.VMEM_SHARED((vpad, _D), f32),
        pltpu.VMEM_SHARED((vpad, 16), f32),
        pltpu.SemaphoreType.DMA,
        pltpu.SemaphoreType.DMA,
        pltpu.SemaphoreType.DMA,
        pltpu.SemaphoreType.DMA,
        pltpu.SemaphoreType.DMA,
        pltpu.SemaphoreType.DMA,
        pltpu.SemaphoreType.DMA,
        pltpu.SemaphoreType.DMA,
    ]


def _make_sc_nodes():
    mesh = plsc.VectorSubcoreMesh(core_axis_name="c", subcore_axis_name="s",
                                  num_cores=_NC, num_subcores=_NS)
    f32 = jnp.float32
    out_type = (
        jax.ShapeDtypeStruct((_NC, _NPAD, _D), f32),
        jax.ShapeDtypeStruct((_NC, _NPAD, 16), f32),
        jax.ShapeDtypeStruct((_NC, _NPAD, _D), f32),
        jax.ShapeDtypeStruct((_NC, _NPAD, 16), f32),
        jax.ShapeDtypeStruct((_NC, _HPAD, _D), f32),
        jax.ShapeDtypeStruct((_NC, _HPAD, 16), f32),
    )
    return pl.kernel(_sc_kernel_nodes, out_type=out_type, mesh=mesh,
                     scratch_types=_sc_scratch(_NPAD),
                     compiler_params=pltpu.CompilerParams(
                         use_tc_tiling_on_sc=False))


def _make_sc_n2h():
    mesh = plsc.VectorSubcoreMesh(core_axis_name="c", subcore_axis_name="s",
                                  num_cores=_NC, num_subcores=_NS)
    f32 = jnp.float32
    out_type = (
        jax.ShapeDtypeStruct((_NC, _HPAD, _D), f32),
        jax.ShapeDtypeStruct((_NC, _HPAD, 16), f32),
    )
    return pl.kernel(_sc_kernel_n2h, out_type=out_type, mesh=mesh,
                     scratch_types=_sc_scratch(_HPAD),
                     compiler_params=pltpu.CompilerParams(
                         use_tc_tiling_on_sc=False))


def _tc_combine_body(a_m, c_m, a_s, c_s, wt_m, b_m, wt_s, b_s, out):
    am = a_m[0] + a_m[1]
    cm = (c_m[0] + c_m[1])[:, 0:1]
    gm = jnp.dot(am, wt_m[...], preferred_element_type=jnp.float32,
                 precision=lax.Precision.HIGHEST)
    gm = gm + cm * b_m[...]
    as_ = a_s[0] + a_s[1]
    cs = (c_s[0] + c_s[1])[:, 0:1]
    gs = jnp.dot(as_, wt_s[...], preferred_element_type=jnp.float32,
                 precision=lax.Precision.HIGHEST)
    gs = gs + cs * b_s[...]
    out[...] = jnp.tanh(gs * gm)


def _tc_combine(a_m, c_m, a_s, c_s, wt_m, b_m, wt_s, b_s, vpad):
    grid = vpad // _BR
    blk_a = pl.BlockSpec((_NC, _BR, _D), lambda i: (0, i, 0))
    blk_c = pl.BlockSpec((_NC, _BR, 16), lambda i: (0, i, 0))
    blk_w = pl.BlockSpec((_D, _D), lambda i: (0, 0))
    blk_b = pl.BlockSpec((1, _D), lambda i: (0, 0))
    return pl.pallas_call(
        _tc_combine_body,
        grid=(grid,),
        in_specs=[blk_a, blk_c, blk_a, blk_c, blk_w, blk_b, blk_w, blk_b],
        out_specs=pl.BlockSpec((_BR, _D), lambda i: (i, 0)),
        out_shape=jax.ShapeDtypeStruct((vpad, _D), jnp.float32),
    )(a_m, c_m, a_s, c_s, wt_m, b_m, wt_s, b_s)


def kernel(node_features, hedge_features, node_senders, node_receivers,
           node_convolution, hedge2node_senders, hedge2node_receivers,
           hedge2node_convolution, hedge_senders, hedge_receivers,
           hedge_convolution, node2hedge_senders, node2hedge_receivers,
           node2hedge_convolution, W_nm, b_nm, W_hs, b_hs, W_hm, b_hm,
           W_ns, b_ns):
    e_nn, c_nn = _prep_edges(node_senders, node_receivers, node_convolution)
    e_h2, c_h2 = _prep_edges(hedge2node_senders, hedge2node_receivers,
                             hedge2node_convolution)
    e_hh, c_hh = _prep_edges(hedge_senders, hedge_receivers, hedge_convolution)
    e_nh, c_nh = _prep_edges(node2hedge_senders, node2hedge_receivers,
                             node2hedge_convolution)

    a_nn, c1_nn, a_h2, c1_h2, a_hh, c1_hh = _make_sc_nodes()(
        node_features, hedge_features, e_nn, c_nn, e_h2, c_h2, e_hh, c_hh)

    new_node_full = _tc_combine(a_h2, c1_h2, a_nn, c1_nn,
                                W_hs.T, b_hs.reshape(1, _D),
                                W_nm.T, b_nm.reshape(1, _D), _NPAD)

    a_nh, c1_nh = _make_sc_n2h()(new_node_full, e_nh, c_nh)

    new_hedge_full = _tc_combine(a_nh, c1_nh, a_hh, c1_hh,
                                 W_ns.T, b_ns.reshape(1, _D),
                                 W_hm.T, b_hm.reshape(1, _D), _HPAD)

    return new_node_full[:_N], new_hedge_full[:_H]
